# baseline (speedup 1.0000x reference)
import functools

import jax
import jax.numpy as jnp
from jax import lax
from jax.experimental import pallas as pl
from jax.experimental.pallas import tpu as pltpu
from jax.experimental.pallas import tpu_sc as plsc

BLK = 1000

NC = 2
NS = 16
CH = 125
NBUF = 5


def _leaky(x):
    return jnp.where(x >= 0, x, 0.01 * x)


def _row_spec(width):
    return pl.BlockSpec((BLK, width), lambda i: (i, 0))


def _full_spec(shape):
    return pl.BlockSpec(shape, lambda i: tuple(0 for _ in shape))


def _part_spec(width):
    return pl.BlockSpec((NC, BLK, width), lambda i: (0, i, 0))


def _tc1_body(des, num, cat, Wd, bd, Wn, bn, Wc, bc, Wdi, Wni, Wci, bi,
              Wl1p, B1p, Wr1, b1, y1p_ref, r1_ref):
    d = _leaky(jnp.dot(des[...], Wd[...], preferred_element_type=jnp.float32) + bd[...])
    n = _leaky(jnp.dot(num[...], Wn[...], preferred_element_type=jnp.float32) + bn[...])
    c = _leaky(jnp.dot(cat[...], Wc[...], preferred_element_type=jnp.float32) + bc[...])
    x = _leaky(jnp.dot(d, Wdi[...], preferred_element_type=jnp.float32)
               + jnp.dot(n, Wni[...], preferred_element_type=jnp.float32)
               + jnp.dot(c, Wci[...], preferred_element_type=jnp.float32)
               + bi[...])
    y1p_ref[...] = jnp.dot(x, Wl1p[...], preferred_element_type=jnp.float32) + B1p[...]
    r1_ref[...] = jnp.dot(x, Wr1[...], preferred_element_type=jnp.float32) + b1[...]


def _tc1(n_nodes, des, num, cat, Wd, bd, Wn, bn, Wc, bc, Wdi, Wni, Wci, bi,
         Wl1p, B1p, Wr1, b1):
    grid = (n_nodes // BLK,)
    return pl.pallas_call(
        _tc1_body,
        grid=grid,
        in_specs=[
            _row_spec(768), _row_spec(4), _row_spec(3),
            _full_spec((768, 32)), _full_spec((1, 32)),
            _full_spec((4, 42)), _full_spec((1, 42)),
            _full_spec((3, 42)), _full_spec((1, 42)),
            _full_spec((32, 128)), _full_spec((42, 128)), _full_spec((42, 128)),
            _full_spec((1, 128)),
            _full_spec((128, 80)), _full_spec((1, 80)),
            _full_spec((128, 64)), _full_spec((1, 64)),
        ],
        out_specs=[_row_spec(80), _row_spec(64)],
        out_shape=[
            jax.ShapeDtypeStruct((n_nodes, 80), jnp.float32),
            jax.ShapeDtypeStruct((n_nodes, 64), jnp.float32),
        ],
    )(des, num, cat, Wd, bd, Wn, bn, Wc, bc, Wdi, Wni, Wci, bi,
      Wl1p, B1p, Wr1, b1)


def _tc2_body(p, r1, Wr2, b2, h1_ref, r2_ref, rc_ref):
    agg = p[0] + p[1]
    rc = 1.0 / jnp.maximum(agg[:, 64:65], 1.0)
    h = jnp.maximum(agg[:, :64] * rc + r1[...], 0.0)
    h1_ref[...] = h
    r2_ref[...] = jnp.dot(h, Wr2[...], preferred_element_type=jnp.float32) + b2[...]
    rc_ref[...] = rc


def _tc2(n_nodes, p, r1, Wr2, b2):
    return pl.pallas_call(
        _tc2_body,
        grid=(n_nodes // BLK,),
        in_specs=[_part_spec(80), _row_spec(64),
                  _full_spec((64, 128)), _full_spec((1, 128))],
        out_specs=[_row_spec(64), _row_spec(128), _row_spec(1)],
        out_shape=[
            jax.ShapeDtypeStruct((n_nodes, 64), jnp.float32),
            jax.ShapeDtypeStruct((n_nodes, 128), jnp.float32),
            jax.ShapeDtypeStruct((n_nodes, 1), jnp.float32),
        ],
    )(p, r1, Wr2, b2)


def _tc3_body(p, rc, r2, Wl2, Wl3, Wr3, b3, y3_ref, r3_ref):
    agg = (p[0] + p[1]) * rc[...]
    x2 = jnp.maximum(jnp.dot(agg, Wl2[...], preferred_element_type=jnp.float32)
                     + r2[...], 0.0)
    y3_ref[...] = jnp.dot(x2, Wl3[...], preferred_element_type=jnp.float32)
    r3_ref[...] = jnp.dot(x2, Wr3[...], preferred_element_type=jnp.float32) + b3[...]


def _tc3(n_nodes, p, rc, r2, Wl2, Wl3, Wr3, b3):
    return pl.pallas_call(
        _tc3_body,
        grid=(n_nodes // BLK,),
        in_specs=[_part_spec(64), _row_spec(1), _row_spec(128),
                  _full_spec((64, 128)), _full_spec((128, 64)),
                  _full_spec((128, 64)), _full_spec((1, 64))],
        out_specs=[_row_spec(64), _row_spec(64)],
        out_shape=[
            jax.ShapeDtypeStruct((n_nodes, 64), jnp.float32),
            jax.ShapeDtypeStruct((n_nodes, 64), jnp.float32),
        ],
    )(p, rc, r2, Wl2, Wl3, Wr3, b3)


def _tc4_body(p, rc, r3, Wr4, b4, h3_ref, r4_ref):
    h = jnp.maximum((p[0] + p[1]) * rc[...] + r3[...], 0.0)
    h3_ref[...] = h
    r4_ref[...] = jnp.dot(h, Wr4[...], preferred_element_type=jnp.float32) + b4[...]


def _tc4(n_nodes, p, rc, r3, Wr4, b4):
    return pl.pallas_call(
        _tc4_body,
        grid=(n_nodes // BLK,),
        in_specs=[_part_spec(64), _row_spec(1), _row_spec(64),
                  _full_spec((64, 128)), _full_spec((1, 128))],
        out_specs=[_row_spec(64), _row_spec(128)],
        out_shape=[
            jax.ShapeDtypeStruct((n_nodes, 64), jnp.float32),
            jax.ShapeDtypeStruct((n_nodes, 128), jnp.float32),
        ],
    )(p, rc, r3, Wr4, b4)


def _tc5_body(p, rc, r4, Wl4, Wo1, bo1, Wo2, bo2, out_ref):
    agg = (p[0] + p[1]) * rc[...]
    x4 = jnp.maximum(jnp.dot(agg, Wl4[...], preferred_element_type=jnp.float32)
                     + r4[...], 0.0)
    z = _leaky(jnp.dot(x4, Wo1[...], preferred_element_type=jnp.float32) + bo1[...])
    out_ref[...] = jnp.dot(z, Wo2[...], preferred_element_type=jnp.float32) + bo2[...]


def _tc5(n_nodes, p, rc, r4, Wl4, Wo1, bo1, Wo2, bo2):
    return pl.pallas_call(
        _tc5_body,
        grid=(n_nodes // BLK,),
        in_specs=[_part_spec(64), _row_spec(1), _row_spec(128),
                  _full_spec((64, 128)), _full_spec((128, 128)),
                  _full_spec((1, 128)), _full_spec((128, 2)), _full_spec((1, 2))],
        out_specs=[_row_spec(2)],
        out_shape=[jax.ShapeDtypeStruct((n_nodes, 2), jnp.float32)],
    )(p, rc, r4, Wl4, Wo1, bo1, Wo2, bo2)[0]


@functools.lru_cache(maxsize=None)
def _make_sc_agg(n_nodes, width, n_edges):
    epc = n_edges // (NC * NS)
    n_ch = epc // CH
    n_rounds = n_ch // NBUF
    npad = -(-n_nodes // (NS * 8)) * (NS * 8)
    rpt = npad // NS
    mesh = plsc.VectorSubcoreMesh(core_axis_name="c", subcore_axis_name="s",
                                  num_cores=NC, num_subcores=NS)

    @functools.partial(
        pl.kernel,
        mesh=mesh,
        out_type=jax.ShapeDtypeStruct((NC, npad, width), jnp.float32),
        scratch_types=(
            [pltpu.VMEM((n_ch, CH), jnp.int32)] * 2
            + [pltpu.VMEM((CH, width), jnp.float32)] * NBUF
            + [pltpu.VMEM_SHARED((npad, width), jnp.float32)]
            + [pltpu.SemaphoreType.DMA] * (NBUF + 1)
        ),
        compiler_params=pltpu.CompilerParams(use_tc_tiling_on_sc=False),
    )
    def agg(x_hbm, edges_hbm, zeros_hbm, out_hbm, *scr):
        sidx, didx = scr[0], scr[1]
        rows = scr[2:2 + NBUF]
        acc = scr[2 + NBUF]
        gsem = scr[3 + NBUF:3 + 2 * NBUF]
        isem = scr[3 + 2 * NBUF]
        c = lax.axis_index("c")
        s = lax.axis_index("s")
        tile = c * NS + s
        chunk0 = tile * n_ch
        row0 = s * rpt

        pltpu.async_copy(edges_hbm.at[0, pl.ds(chunk0, n_ch)], sidx, isem)
        pltpu.async_copy(edges_hbm.at[1, pl.ds(chunk0, n_ch)], didx, isem)
        pltpu.sync_copy(zeros_hbm.at[pl.ds(row0, rpt)], acc.at[pl.ds(row0, rpt)])
        pltpu.make_async_copy(edges_hbm.at[0, pl.ds(chunk0, n_ch)], sidx, isem).wait()
        pltpu.make_async_copy(edges_hbm.at[1, pl.ds(chunk0, n_ch)], didx, isem).wait()
        plsc.subcore_barrier()

        def gather(i, b):
            return pltpu.async_copy(x_hbm.at[sidx.at[i]], rows[b], gsem[b])

        def scatter(i, b):
            pltpu.make_async_copy(x_hbm.at[sidx.at[i]], rows[b], gsem[b]).wait()
            pltpu.sync_copy(rows[b], acc.at[didx.at[i]], add=True)

        for b in range(NBUF):
            gather(b, b)

        def body(r, _):
            i0 = r * NBUF
            for b in range(NBUF):
                scatter(i0 + b, b)
                gather(i0 + NBUF + b, b)
            return 0

        lax.fori_loop(0, n_rounds - 1, body, 0)
        i0 = (n_rounds - 1) * NBUF
        for b in range(NBUF):
            scatter(i0 + b, b)

        plsc.subcore_barrier()
        pltpu.sync_copy(acc.at[pl.ds(row0, rpt)],
                        out_hbm.at[c, pl.ds(row0, rpt)])

    return agg


def _sc_agg(x, edges3):
    n_nodes, width = x.shape
    n_edges = edges3.shape[1] * edges3.shape[2]
    npad = -(-n_nodes // (NS * 8)) * (NS * 8)
    zeros = jnp.zeros((npad, width), jnp.float32)
    return _make_sc_agg(n_nodes, width, n_edges)(x, edges3, zeros)


def kernel(des, tweet, num_prop, cat_prop, edge_index, W_des, b_des, W_num,
           b_num, W_cat, b_cat, W_in, b_in, s1a_Wl, s1a_Wr, s1a_b, s1b_Wl,
           s1b_Wr, s1b_b, s2a_Wl, s2a_Wr, s2a_b, s2b_Wl, s2b_Wr, s2b_b,
           W_o1, b_o1, W_o2, b_o2):
    n_nodes = des.shape[0]
    n_edges = edge_index.shape[1]
    edges3 = edge_index.reshape(2, n_edges // CH, CH)

    Wl1p = jnp.concatenate([s1a_Wl, jnp.zeros((128, 16), jnp.float32)], axis=1)
    B1p = jnp.concatenate([jnp.zeros((1, 64), jnp.float32),
                           jnp.ones((1, 16), jnp.float32)], axis=1)

    y1p, r1 = _tc1(
        n_nodes, des, num_prop, cat_prop,
        W_des, b_des.reshape(1, -1), W_num, b_num.reshape(1, -1),
        W_cat, b_cat.reshape(1, -1),
        W_in[:32], W_in[32:74], W_in[74:116], b_in.reshape(1, -1),
        Wl1p, B1p, s1a_Wr, s1a_b.reshape(1, -1))

    p1 = _sc_agg(y1p, edges3)
    h1, r2, rc = _tc2(n_nodes, p1, r1, s1b_Wr, s1b_b.reshape(1, -1))

    p2 = _sc_agg(h1, edges3)
    y3, r3 = _tc3(n_nodes, p2, rc, r2, s1b_Wl, s2a_Wl, s2a_Wr,
                  s2a_b.reshape(1, -1))

    p3 = _sc_agg(y3, edges3)
    h3, r4 = _tc4(n_nodes, p3, rc, r3, s2b_Wr, s2b_b.reshape(1, -1))

    p4 = _sc_agg(h3, edges3)
    out = _tc5(n_nodes, p4, rc, r4, s2b_Wl, W_o1, b_o1.reshape(1, -1),
               W_o2, b_o2.reshape(1, -1))
    return out


# speedup vs baseline: 1.0198x; 1.0198x over previous
import functools

import jax
import jax.numpy as jnp
from jax import lax
from jax.experimental import pallas as pl
from jax.experimental.pallas import tpu as pltpu
from jax.experimental.pallas import tpu_sc as plsc

BLK = 2000

NC = 2
NS = 16
CH = 125
NBUF = 5


def _leaky(x):
    return jnp.where(x >= 0, x, 0.01 * x)


def _row_spec(width):
    return pl.BlockSpec((BLK, width), lambda i: (i, 0))


def _full_spec(shape):
    return pl.BlockSpec(shape, lambda i: tuple(0 for _ in shape))


def _part_spec(width):
    return pl.BlockSpec((NC, BLK, width), lambda i: (0, i, 0))


def _tc1_body(des, num, cat, Wd, bd, Wn, bn, Wc, bc, Wdi, Wni, Wci, bi,
              Wl1p, B1p, Wr1, b1, y1p_ref, r1_ref):
    d = _leaky(jnp.dot(des[...], Wd[...], preferred_element_type=jnp.float32) + bd[...])
    n = _leaky(jnp.dot(num[...], Wn[...], preferred_element_type=jnp.float32) + bn[...])
    c = _leaky(jnp.dot(cat[...], Wc[...], preferred_element_type=jnp.float32) + bc[...])
    x = _leaky(jnp.dot(d, Wdi[...], preferred_element_type=jnp.float32)
               + jnp.dot(n, Wni[...], preferred_element_type=jnp.float32)
               + jnp.dot(c, Wci[...], preferred_element_type=jnp.float32)
               + bi[...])
    y1p_ref[...] = jnp.dot(x, Wl1p[...], preferred_element_type=jnp.float32) + B1p[...]
    r1_ref[...] = jnp.dot(x, Wr1[...], preferred_element_type=jnp.float32) + b1[...]


def _tc1(n_nodes, des, num, cat, Wd, bd, Wn, bn, Wc, bc, Wdi, Wni, Wci, bi,
         Wl1p, B1p, Wr1, b1):
    grid = (n_nodes // BLK,)
    return pl.pallas_call(
        _tc1_body,
        grid=grid,
        in_specs=[
            _row_spec(768), _row_spec(4), _row_spec(3),
            _full_spec((768, 32)), _full_spec((1, 32)),
            _full_spec((4, 42)), _full_spec((1, 42)),
            _full_spec((3, 42)), _full_spec((1, 42)),
            _full_spec((32, 128)), _full_spec((42, 128)), _full_spec((42, 128)),
            _full_spec((1, 128)),
            _full_spec((128, 80)), _full_spec((1, 80)),
            _full_spec((128, 64)), _full_spec((1, 64)),
        ],
        out_specs=[_row_spec(80), _row_spec(64)],
        out_shape=[
            jax.ShapeDtypeStruct((n_nodes, 80), jnp.float32),
            jax.ShapeDtypeStruct((n_nodes, 64), jnp.float32),
        ],
    )(des, num, cat, Wd, bd, Wn, bn, Wc, bc, Wdi, Wni, Wci, bi,
      Wl1p, B1p, Wr1, b1)


def _tc2_body(p, r1, h1_ref, rc_ref):
    agg = p[0] + p[1]
    rc = 1.0 / jnp.maximum(agg[:, 64:65], 1.0)
    h = jnp.maximum(agg[:, :64] * rc + r1[...], 0.0)
    h1_ref[...] = h
    rc_ref[...] = rc


def _tc2(n_nodes, p, r1):
    return pl.pallas_call(
        _tc2_body,
        grid=(n_nodes // BLK,),
        in_specs=[_part_spec(80), _row_spec(64)],
        out_specs=[_row_spec(64), _row_spec(1)],
        out_shape=[
            jax.ShapeDtypeStruct((n_nodes, 64), jnp.float32),
            jax.ShapeDtypeStruct((n_nodes, 1), jnp.float32),
        ],
    )(p, r1)


def _tcmm_body(h, W, b, r_ref):
    r_ref[...] = jnp.dot(h[...], W[...], preferred_element_type=jnp.float32) + b[...]


def _tcmm(n_nodes, h, W, b):
    k, m = W.shape
    return pl.pallas_call(
        _tcmm_body,
        grid=(n_nodes // BLK,),
        in_specs=[_row_spec(k), _full_spec((k, m)), _full_spec((1, m))],
        out_specs=[_row_spec(m)],
        out_shape=[jax.ShapeDtypeStruct((n_nodes, m), jnp.float32)],
    )(h, W, b)[0]


def _tc3_body(p, rc, r2, Wl2, Wl3, y3_ref, x2_ref):
    agg = (p[0] + p[1]) * rc[...]
    x2 = jnp.maximum(jnp.dot(agg, Wl2[...], preferred_element_type=jnp.float32)
                     + r2[...], 0.0)
    y3_ref[...] = jnp.dot(x2, Wl3[...], preferred_element_type=jnp.float32)
    x2_ref[...] = x2


def _tc3(n_nodes, p, rc, r2, Wl2, Wl3):
    return pl.pallas_call(
        _tc3_body,
        grid=(n_nodes // BLK,),
        in_specs=[_part_spec(64), _row_spec(1), _row_spec(128),
                  _full_spec((64, 128)), _full_spec((128, 64))],
        out_specs=[_row_spec(64), _row_spec(128)],
        out_shape=[
            jax.ShapeDtypeStruct((n_nodes, 64), jnp.float32),
            jax.ShapeDtypeStruct((n_nodes, 128), jnp.float32),
        ],
    )(p, rc, r2, Wl2, Wl3)


def _tc4_body(p, rc, r3, h3_ref):
    h3_ref[...] = jnp.maximum((p[0] + p[1]) * rc[...] + r3[...], 0.0)


def _tc4(n_nodes, p, rc, r3):
    return pl.pallas_call(
        _tc4_body,
        grid=(n_nodes // BLK,),
        in_specs=[_part_spec(64), _row_spec(1), _row_spec(64)],
        out_specs=[_row_spec(64)],
        out_shape=[jax.ShapeDtypeStruct((n_nodes, 64), jnp.float32)],
    )(p, rc, r3)[0]


def _tc5_body(p, rc, r4, Wl4, Wo1, bo1, Wo2, bo2, out_ref):
    agg = (p[0] + p[1]) * rc[...]
    x4 = jnp.maximum(jnp.dot(agg, Wl4[...], preferred_element_type=jnp.float32)
                     + r4[...], 0.0)
    z = _leaky(jnp.dot(x4, Wo1[...], preferred_element_type=jnp.float32) + bo1[...])
    out_ref[...] = jnp.dot(z, Wo2[...], preferred_element_type=jnp.float32) + bo2[...]


def _tc5(n_nodes, p, rc, r4, Wl4, Wo1, bo1, Wo2, bo2):
    return pl.pallas_call(
        _tc5_body,
        grid=(n_nodes // BLK,),
        in_specs=[_part_spec(64), _row_spec(1), _row_spec(128),
                  _full_spec((64, 128)), _full_spec((128, 128)),
                  _full_spec((1, 128)), _full_spec((128, 2)), _full_spec((1, 2))],
        out_specs=[_row_spec(2)],
        out_shape=[jax.ShapeDtypeStruct((n_nodes, 2), jnp.float32)],
    )(p, rc, r4, Wl4, Wo1, bo1, Wo2, bo2)[0]


@functools.lru_cache(maxsize=None)
def _make_sc_agg(n_nodes, width, n_edges):
    epc = n_edges // (NC * NS)
    n_ch = epc // CH
    n_rounds = n_ch // NBUF
    npad = -(-n_nodes // (NS * 8)) * (NS * 8)
    rpt = npad // NS
    mesh = plsc.VectorSubcoreMesh(core_axis_name="c", subcore_axis_name="s",
                                  num_cores=NC, num_subcores=NS)

    @functools.partial(
        pl.kernel,
        mesh=mesh,
        out_type=jax.ShapeDtypeStruct((NC, npad, width), jnp.float32),
        scratch_types=(
            [pltpu.VMEM((n_ch, CH), jnp.int32)] * 2
            + [pltpu.VMEM((CH, width), jnp.float32)] * NBUF
            + [pltpu.VMEM_SHARED((npad, width), jnp.float32)]
            + [pltpu.SemaphoreType.DMA] * (NBUF + 1)
        ),
        compiler_params=pltpu.CompilerParams(use_tc_tiling_on_sc=False),
    )
    def agg(x_hbm, edges_hbm, zeros_hbm, out_hbm, *scr):
        sidx, didx = scr[0], scr[1]
        rows = scr[2:2 + NBUF]
        acc = scr[2 + NBUF]
        gsem = scr[3 + NBUF:3 + 2 * NBUF]
        isem = scr[3 + 2 * NBUF]
        c = lax.axis_index("c")
        s = lax.axis_index("s")
        tile = c * NS + s
        chunk0 = tile * n_ch
        row0 = s * rpt

        pltpu.async_copy(edges_hbm.at[0, pl.ds(chunk0, n_ch)], sidx, isem)
        pltpu.async_copy(edges_hbm.at[1, pl.ds(chunk0, n_ch)], didx, isem)
        pltpu.sync_copy(zeros_hbm.at[pl.ds(row0, rpt)], acc.at[pl.ds(row0, rpt)])
        pltpu.make_async_copy(edges_hbm.at[0, pl.ds(chunk0, n_ch)], sidx, isem).wait()
        pltpu.make_async_copy(edges_hbm.at[1, pl.ds(chunk0, n_ch)], didx, isem).wait()
        plsc.subcore_barrier()

        def gather(i, b):
            return pltpu.async_copy(x_hbm.at[sidx.at[i]], rows[b], gsem[b])

        def scatter(i, b):
            pltpu.make_async_copy(x_hbm.at[sidx.at[i]], rows[b], gsem[b]).wait()
            pltpu.sync_copy(rows[b], acc.at[didx.at[i]], add=True)

        for b in range(NBUF):
            gather(b, b)

        def body(r, _):
            i0 = r * NBUF
            for b in range(NBUF):
                scatter(i0 + b, b)
                gather(i0 + NBUF + b, b)
            return 0

        lax.fori_loop(0, n_rounds - 1, body, 0)
        i0 = (n_rounds - 1) * NBUF
        for b in range(NBUF):
            scatter(i0 + b, b)

        plsc.subcore_barrier()
        pltpu.sync_copy(acc.at[pl.ds(row0, rpt)],
                        out_hbm.at[c, pl.ds(row0, rpt)])

    return agg


def _sc_agg(x, edges3):
    n_nodes, width = x.shape
    n_edges = edges3.shape[1] * edges3.shape[2]
    npad = -(-n_nodes // (NS * 8)) * (NS * 8)
    zeros = jnp.zeros((npad, width), jnp.float32)
    return _make_sc_agg(n_nodes, width, n_edges)(x, edges3, zeros)


def kernel(des, tweet, num_prop, cat_prop, edge_index, W_des, b_des, W_num,
           b_num, W_cat, b_cat, W_in, b_in, s1a_Wl, s1a_Wr, s1a_b, s1b_Wl,
           s1b_Wr, s1b_b, s2a_Wl, s2a_Wr, s2a_b, s2b_Wl, s2b_Wr, s2b_b,
           W_o1, b_o1, W_o2, b_o2):
    n_nodes = des.shape[0]
    n_edges = edge_index.shape[1]
    edges3 = edge_index.reshape(2, n_edges // CH, CH)

    Wl1p = jnp.concatenate([s1a_Wl, jnp.zeros((128, 16), jnp.float32)], axis=1)
    B1p = jnp.concatenate([jnp.zeros((1, 64), jnp.float32),
                           jnp.ones((1, 16), jnp.float32)], axis=1)

    y1p, r1 = _tc1(
        n_nodes, des, num_prop, cat_prop,
        W_des, b_des.reshape(1, -1), W_num, b_num.reshape(1, -1),
        W_cat, b_cat.reshape(1, -1),
        W_in[:32], W_in[32:74], W_in[74:116], b_in.reshape(1, -1),
        Wl1p, B1p, s1a_Wr, s1a_b.reshape(1, -1))

    p1 = _sc_agg(y1p, edges3)
    h1, rc = _tc2(n_nodes, p1, r1)

    p2 = _sc_agg(h1, edges3)
    r2 = _tcmm(n_nodes, h1, s1b_Wr, s1b_b.reshape(1, -1))
    y3, x2 = _tc3(n_nodes, p2, rc, r2, s1b_Wl, s2a_Wl)

    p3 = _sc_agg(y3, edges3)
    r3 = _tcmm(n_nodes, x2, s2a_Wr, s2a_b.reshape(1, -1))
    h3 = _tc4(n_nodes, p3, rc, r3)

    p4 = _sc_agg(h3, edges3)
    r4 = _tcmm(n_nodes, h3, s2b_Wr, s2b_b.reshape(1, -1))
    out = _tc5(n_nodes, p4, rc, r4, s2b_Wl, W_o1, b_o1.reshape(1, -1),
               W_o2, b_o2.reshape(1, -1))
    return out


# speedup vs baseline: 1.0247x; 1.0048x over previous
import functools

import jax
import jax.numpy as jnp
from jax import lax
from jax.experimental import pallas as pl
from jax.experimental.pallas import tpu as pltpu
from jax.experimental.pallas import tpu_sc as plsc

BLK = 2000

NC = 2
NS = 16
CH = 125
NBUF = 5


def _leaky(x):
    return jnp.where(x >= 0, x, 0.01 * x)


def _row_spec(width):
    return pl.BlockSpec((BLK, width), lambda i: (i, 0))


def _full_spec(shape):
    return pl.BlockSpec(shape, lambda i: tuple(0 for _ in shape))


def _part_spec(width):
    return pl.BlockSpec((NC, BLK, width), lambda i: (0, i, 0))


def _tc1_body(des, num, cat, Wd, bd, Wn, bn, Wc, bc, Wdi, Wni, Wci, bi,
              Wl1p, B1p, Wr1, b1, y1p_ref, r1_ref):
    fast = jax.lax.Precision.DEFAULT
    d = _leaky(jnp.dot(des[...], Wd[...], precision=fast, preferred_element_type=jnp.float32) + bd[...])
    n = _leaky(jnp.dot(num[...], Wn[...], preferred_element_type=jnp.float32) + bn[...])
    c = _leaky(jnp.dot(cat[...], Wc[...], preferred_element_type=jnp.float32) + bc[...])
    x = _leaky(jnp.dot(d, Wdi[...], preferred_element_type=jnp.float32)
               + jnp.dot(n, Wni[...], preferred_element_type=jnp.float32)
               + jnp.dot(c, Wci[...], preferred_element_type=jnp.float32)
               + bi[...])
    y1p_ref[...] = jnp.dot(x, Wl1p[...], preferred_element_type=jnp.float32) + B1p[...]
    r1_ref[...] = jnp.dot(x, Wr1[...], preferred_element_type=jnp.float32) + b1[...]


def _tc1(n_nodes, des, num, cat, Wd, bd, Wn, bn, Wc, bc, Wdi, Wni, Wci, bi,
         Wl1p, B1p, Wr1, b1):
    grid = (n_nodes // BLK,)
    return pl.pallas_call(
        _tc1_body,
        grid=grid,
        in_specs=[
            _row_spec(768), _row_spec(4), _row_spec(3),
            _full_spec((768, 32)), _full_spec((1, 32)),
            _full_spec((4, 42)), _full_spec((1, 42)),
            _full_spec((3, 42)), _full_spec((1, 42)),
            _full_spec((32, 128)), _full_spec((42, 128)), _full_spec((42, 128)),
            _full_spec((1, 128)),
            _full_spec((128, 80)), _full_spec((1, 80)),
            _full_spec((128, 64)), _full_spec((1, 64)),
        ],
        out_specs=[_row_spec(80), _row_spec(64)],
        out_shape=[
            jax.ShapeDtypeStruct((n_nodes, 80), jnp.float32),
            jax.ShapeDtypeStruct((n_nodes, 64), jnp.float32),
        ],
    )(des, num, cat, Wd, bd, Wn, bn, Wc, bc, Wdi, Wni, Wci, bi,
      Wl1p, B1p, Wr1, b1)


def _tc2_body(p, r1, Wr2, b2, h1_ref, r2_ref, rc_ref):
    agg = p[0] + p[1]
    rc = 1.0 / jnp.maximum(agg[:, 64:65], 1.0)
    h = jnp.maximum(agg[:, :64] * rc + r1[...], 0.0)
    h1_ref[...] = h
    r2_ref[...] = jnp.dot(h, Wr2[...], preferred_element_type=jnp.float32) + b2[...]
    rc_ref[...] = rc


def _tc2(n_nodes, p, r1, Wr2, b2):
    return pl.pallas_call(
        _tc2_body,
        grid=(n_nodes // BLK,),
        in_specs=[_part_spec(80), _row_spec(64),
                  _full_spec((64, 128)), _full_spec((1, 128))],
        out_specs=[_row_spec(64), _row_spec(128), _row_spec(1)],
        out_shape=[
            jax.ShapeDtypeStruct((n_nodes, 64), jnp.float32),
            jax.ShapeDtypeStruct((n_nodes, 128), jnp.float32),
            jax.ShapeDtypeStruct((n_nodes, 1), jnp.float32),
        ],
    )(p, r1, Wr2, b2)


def _tc3_body(p, rc, r2, Wl2, Wl3, Wr3, b3, y3_ref, r3_ref):
    agg = (p[0] + p[1]) * rc[...]
    x2 = jnp.maximum(jnp.dot(agg, Wl2[...], preferred_element_type=jnp.float32)
                     + r2[...], 0.0)
    y3_ref[...] = jnp.dot(x2, Wl3[...], preferred_element_type=jnp.float32)
    r3_ref[...] = jnp.dot(x2, Wr3[...], preferred_element_type=jnp.float32) + b3[...]


def _tc3(n_nodes, p, rc, r2, Wl2, Wl3, Wr3, b3):
    return pl.pallas_call(
        _tc3_body,
        grid=(n_nodes // BLK,),
        in_specs=[_part_spec(64), _row_spec(1), _row_spec(128),
                  _full_spec((64, 128)), _full_spec((128, 64)),
                  _full_spec((128, 64)), _full_spec((1, 64))],
        out_specs=[_row_spec(64), _row_spec(64)],
        out_shape=[
            jax.ShapeDtypeStruct((n_nodes, 64), jnp.float32),
            jax.ShapeDtypeStruct((n_nodes, 64), jnp.float32),
        ],
    )(p, rc, r2, Wl2, Wl3, Wr3, b3)


def _tc4_body(p, rc, r3, Wr4, b4, h3_ref, r4_ref):
    h = jnp.maximum((p[0] + p[1]) * rc[...] + r3[...], 0.0)
    h3_ref[...] = h
    r4_ref[...] = jnp.dot(h, Wr4[...], preferred_element_type=jnp.float32) + b4[...]


def _tc4(n_nodes, p, rc, r3, Wr4, b4):
    return pl.pallas_call(
        _tc4_body,
        grid=(n_nodes // BLK,),
        in_specs=[_part_spec(64), _row_spec(1), _row_spec(64),
                  _full_spec((64, 128)), _full_spec((1, 128))],
        out_specs=[_row_spec(64), _row_spec(128)],
        out_shape=[
            jax.ShapeDtypeStruct((n_nodes, 64), jnp.float32),
            jax.ShapeDtypeStruct((n_nodes, 128), jnp.float32),
        ],
    )(p, rc, r3, Wr4, b4)


def _tc5_body(p, rc, r4, Wl4, Wo1, bo1, Wo2, bo2, out_ref):
    agg = (p[0] + p[1]) * rc[...]
    x4 = jnp.maximum(jnp.dot(agg, Wl4[...], preferred_element_type=jnp.float32)
                     + r4[...], 0.0)
    z = _leaky(jnp.dot(x4, Wo1[...], preferred_element_type=jnp.float32) + bo1[...])
    out_ref[...] = jnp.dot(z, Wo2[...], preferred_element_type=jnp.float32) + bo2[...]


def _tc5(n_nodes, p, rc, r4, Wl4, Wo1, bo1, Wo2, bo2):
    return pl.pallas_call(
        _tc5_body,
        grid=(n_nodes // BLK,),
        in_specs=[_part_spec(64), _row_spec(1), _row_spec(128),
                  _full_spec((64, 128)), _full_spec((128, 128)),
                  _full_spec((1, 128)), _full_spec((128, 2)), _full_spec((1, 2))],
        out_specs=[_row_spec(2)],
        out_shape=[jax.ShapeDtypeStruct((n_nodes, 2), jnp.float32)],
    )(p, rc, r4, Wl4, Wo1, bo1, Wo2, bo2)[0]


@functools.lru_cache(maxsize=None)
def _make_sc_agg(n_nodes, width, n_edges):
    epc = n_edges // (NC * NS)
    n_ch = epc // CH
    n_rounds = n_ch // NBUF
    npad = -(-n_nodes // (NS * 8)) * (NS * 8)
    rpt = npad // NS
    mesh = plsc.VectorSubcoreMesh(core_axis_name="c", subcore_axis_name="s",
                                  num_cores=NC, num_subcores=NS)

    @functools.partial(
        pl.kernel,
        mesh=mesh,
        out_type=jax.ShapeDtypeStruct((NC, npad, width), jnp.float32),
        scratch_types=(
            [pltpu.VMEM((n_ch, CH), jnp.int32)] * 2
            + [pltpu.VMEM((CH, width), jnp.float32)] * NBUF
            + [pltpu.VMEM_SHARED((npad, width), jnp.float32)]
            + [pltpu.SemaphoreType.DMA] * (NBUF + 1)
        ),
        compiler_params=pltpu.CompilerParams(use_tc_tiling_on_sc=False),
    )
    def agg(x_hbm, edges_hbm, zeros_hbm, out_hbm, *scr):
        sidx, didx = scr[0], scr[1]
        rows = scr[2:2 + NBUF]
        acc = scr[2 + NBUF]
        gsem = scr[3 + NBUF:3 + 2 * NBUF]
        isem = scr[3 + 2 * NBUF]
        c = lax.axis_index("c")
        s = lax.axis_index("s")
        tile = c * NS + s
        chunk0 = tile * n_ch
        row0 = s * rpt

        pltpu.async_copy(edges_hbm.at[0, pl.ds(chunk0, n_ch)], sidx, isem)
        pltpu.async_copy(edges_hbm.at[1, pl.ds(chunk0, n_ch)], didx, isem)
        pltpu.sync_copy(zeros_hbm.at[pl.ds(row0, rpt)], acc.at[pl.ds(row0, rpt)])
        pltpu.make_async_copy(edges_hbm.at[0, pl.ds(chunk0, n_ch)], sidx, isem).wait()
        pltpu.make_async_copy(edges_hbm.at[1, pl.ds(chunk0, n_ch)], didx, isem).wait()
        plsc.subcore_barrier()

        def gather(i, b):
            return pltpu.async_copy(x_hbm.at[sidx.at[i]], rows[b], gsem[b])

        def scatter(i, b):
            pltpu.make_async_copy(x_hbm.at[sidx.at[i]], rows[b], gsem[b]).wait()
            pltpu.sync_copy(rows[b], acc.at[didx.at[i]], add=True)

        for b in range(NBUF):
            gather(b, b)

        def body(r, _):
            i0 = r * NBUF
            for b in range(NBUF):
                scatter(i0 + b, b)
                gather(i0 + NBUF + b, b)
            return 0

        lax.fori_loop(0, n_rounds - 1, body, 0)
        i0 = (n_rounds - 1) * NBUF
        for b in range(NBUF):
            scatter(i0 + b, b)

        plsc.subcore_barrier()
        pltpu.sync_copy(acc.at[pl.ds(row0, rpt)],
                        out_hbm.at[c, pl.ds(row0, rpt)])

    return agg


def _sc_agg(x, edges3):
    n_nodes, width = x.shape
    n_edges = edges3.shape[1] * edges3.shape[2]
    npad = -(-n_nodes // (NS * 8)) * (NS * 8)
    zeros = jnp.zeros((npad, width), jnp.float32)
    return _make_sc_agg(n_nodes, width, n_edges)(x, edges3, zeros)


def kernel(des, tweet, num_prop, cat_prop, edge_index, W_des, b_des, W_num,
           b_num, W_cat, b_cat, W_in, b_in, s1a_Wl, s1a_Wr, s1a_b, s1b_Wl,
           s1b_Wr, s1b_b, s2a_Wl, s2a_Wr, s2a_b, s2b_Wl, s2b_Wr, s2b_b,
           W_o1, b_o1, W_o2, b_o2):
    n_nodes = des.shape[0]
    n_edges = edge_index.shape[1]
    edges3 = edge_index.reshape(2, n_edges // CH, CH)

    Wl1p = jnp.concatenate([s1a_Wl, jnp.zeros((128, 16), jnp.float32)], axis=1)
    B1p = jnp.concatenate([jnp.zeros((1, 64), jnp.float32),
                           jnp.ones((1, 16), jnp.float32)], axis=1)

    y1p, r1 = _tc1(
        n_nodes, des, num_prop, cat_prop,
        W_des, b_des.reshape(1, -1), W_num, b_num.reshape(1, -1),
        W_cat, b_cat.reshape(1, -1),
        W_in[:32], W_in[32:74], W_in[74:116], b_in.reshape(1, -1),
        Wl1p, B1p, s1a_Wr, s1a_b.reshape(1, -1))

    p1 = _sc_agg(y1p, edges3)
    h1, r2, rc = _tc2(n_nodes, p1, r1, s1b_Wr, s1b_b.reshape(1, -1))

    p2 = _sc_agg(h1, edges3)
    y3, r3 = _tc3(n_nodes, p2, rc, r2, s1b_Wl, s2a_Wl, s2a_Wr,
                  s2a_b.reshape(1, -1))

    p3 = _sc_agg(y3, edges3)
    h3, r4 = _tc4(n_nodes, p3, rc, r3, s2b_Wr, s2b_b.reshape(1, -1))

    p4 = _sc_agg(h3, edges3)
    out = _tc5(n_nodes, p4, rc, r4, s2b_Wl, W_o1, b_o1.reshape(1, -1),
               W_o2, b_o2.reshape(1, -1))
    return out


# speedup vs baseline: 1.0578x; 1.0323x over previous
import functools

import jax
import jax.numpy as jnp
from jax import lax
from jax.experimental import pallas as pl
from jax.experimental.pallas import tpu as pltpu
from jax.experimental.pallas import tpu_sc as plsc

BLK = 2000

NC = 2
NS = 16
CH = 125
NBUF = 5


def _leaky(x):
    return jnp.where(x >= 0, x, 0.01 * x)


def _row_spec(width):
    return pl.BlockSpec((BLK, width), lambda i: (i, 0))


def _full_spec(shape):
    return pl.BlockSpec(shape, lambda i: tuple(0 for _ in shape))


def _part_spec(width):
    return pl.BlockSpec((NC, BLK, width), lambda i: (0, i, 0))


def _tc1_body(des, num, cat, Wd, bd, Wn, bn, Wc, bc, Wdi, Wni, Wci, bi,
              Wl1p, B1p, Wr1, b1, y1p_ref, r1_ref):
    d = _leaky(jnp.dot(des[...], Wd[...], preferred_element_type=jnp.float32) + bd[...])
    n = _leaky(jnp.dot(num[...], Wn[...], preferred_element_type=jnp.float32) + bn[...])
    c = _leaky(jnp.dot(cat[...], Wc[...], preferred_element_type=jnp.float32) + bc[...])
    x = _leaky(jnp.dot(d, Wdi[...], preferred_element_type=jnp.float32)
               + jnp.dot(n, Wni[...], preferred_element_type=jnp.float32)
               + jnp.dot(c, Wci[...], preferred_element_type=jnp.float32)
               + bi[...])
    y1p_ref[...] = jnp.dot(x, Wl1p[...], preferred_element_type=jnp.float32) + B1p[...]
    r1_ref[...] = jnp.dot(x, Wr1[...], preferred_element_type=jnp.float32) + b1[...]


def _tc1(n_nodes, des, num, cat, Wd, bd, Wn, bn, Wc, bc, Wdi, Wni, Wci, bi,
         Wl1p, B1p, Wr1, b1):
    grid = (n_nodes // BLK,)
    return pl.pallas_call(
        _tc1_body,
        grid=grid,
        in_specs=[
            _row_spec(768), _row_spec(4), _row_spec(3),
            _full_spec((768, 32)), _full_spec((1, 32)),
            _full_spec((4, 42)), _full_spec((1, 42)),
            _full_spec((3, 42)), _full_spec((1, 42)),
            _full_spec((32, 128)), _full_spec((42, 128)), _full_spec((42, 128)),
            _full_spec((1, 128)),
            _full_spec((128, 80)), _full_spec((1, 80)),
            _full_spec((128, 64)), _full_spec((1, 64)),
        ],
        out_specs=[_row_spec(80), _row_spec(64)],
        out_shape=[
            jax.ShapeDtypeStruct((n_nodes, 80), jnp.float32),
            jax.ShapeDtypeStruct((n_nodes, 64), jnp.float32),
        ],
    )(des, num, cat, Wd, bd, Wn, bn, Wc, bc, Wdi, Wni, Wci, bi,
      Wl1p, B1p, Wr1, b1)


def _tc2_body(p, r1, Wr2, b2, h1_ref, r2_ref, rc_ref):
    agg = p[0] + p[1]
    rc = 1.0 / jnp.maximum(agg[:, 64:65], 1.0)
    h = jnp.maximum(agg[:, :64] * rc + r1[...], 0.0)
    h1_ref[...] = h
    r2_ref[...] = jnp.dot(h, Wr2[...], preferred_element_type=jnp.float32) + b2[...]
    rc_ref[...] = rc


def _tc2(n_nodes, p, r1, Wr2, b2):
    return pl.pallas_call(
        _tc2_body,
        grid=(n_nodes // BLK,),
        in_specs=[_part_spec(80), _row_spec(64),
                  _full_spec((64, 128)), _full_spec((1, 128))],
        out_specs=[_row_spec(64), _row_spec(128), _row_spec(1)],
        out_shape=[
            jax.ShapeDtypeStruct((n_nodes, 64), jnp.float32),
            jax.ShapeDtypeStruct((n_nodes, 128), jnp.float32),
            jax.ShapeDtypeStruct((n_nodes, 1), jnp.float32),
        ],
    )(p, r1, Wr2, b2)


def _tc3_body(p, rc, r2, Wl2, Wl3, Wr3, b3, y3_ref, r3_ref):
    agg = p[...] * rc[...]
    x2 = jnp.maximum(jnp.dot(agg, Wl2[...], preferred_element_type=jnp.float32)
                     + r2[...], 0.0)
    y3_ref[...] = jnp.dot(x2, Wl3[...], preferred_element_type=jnp.float32)
    r3_ref[...] = jnp.dot(x2, Wr3[...], preferred_element_type=jnp.float32) + b3[...]


def _tc3(n_nodes, p, rc, r2, Wl2, Wl3, Wr3, b3):
    return pl.pallas_call(
        _tc3_body,
        grid=(n_nodes // BLK,),
        in_specs=[_row_spec(64), _row_spec(1), _row_spec(128),
                  _full_spec((64, 128)), _full_spec((128, 64)),
                  _full_spec((128, 64)), _full_spec((1, 64))],
        out_specs=[_row_spec(64), _row_spec(64)],
        out_shape=[
            jax.ShapeDtypeStruct((n_nodes, 64), jnp.float32),
            jax.ShapeDtypeStruct((n_nodes, 64), jnp.float32),
        ],
    )(p, rc, r2, Wl2, Wl3, Wr3, b3)


def _tc4_body(p, rc, r3, Wr4, b4, h3_ref, r4_ref):
    h = jnp.maximum(p[...] * rc[...] + r3[...], 0.0)
    h3_ref[...] = h
    r4_ref[...] = jnp.dot(h, Wr4[...], preferred_element_type=jnp.float32) + b4[...]


def _tc4(n_nodes, p, rc, r3, Wr4, b4):
    return pl.pallas_call(
        _tc4_body,
        grid=(n_nodes // BLK,),
        in_specs=[_row_spec(64), _row_spec(1), _row_spec(64),
                  _full_spec((64, 128)), _full_spec((1, 128))],
        out_specs=[_row_spec(64), _row_spec(128)],
        out_shape=[
            jax.ShapeDtypeStruct((n_nodes, 64), jnp.float32),
            jax.ShapeDtypeStruct((n_nodes, 128), jnp.float32),
        ],
    )(p, rc, r3, Wr4, b4)


def _tc5_body(p, rc, r4, Wl4, Wo1, bo1, Wo2, bo2, out_ref):
    agg = p[...] * rc[...]
    x4 = jnp.maximum(jnp.dot(agg, Wl4[...], preferred_element_type=jnp.float32)
                     + r4[...], 0.0)
    z = _leaky(jnp.dot(x4, Wo1[...], preferred_element_type=jnp.float32) + bo1[...])
    out_ref[...] = jnp.dot(z, Wo2[...], preferred_element_type=jnp.float32) + bo2[...]


def _tc5(n_nodes, p, rc, r4, Wl4, Wo1, bo1, Wo2, bo2):
    return pl.pallas_call(
        _tc5_body,
        grid=(n_nodes // BLK,),
        in_specs=[_row_spec(64), _row_spec(1), _row_spec(128),
                  _full_spec((64, 128)), _full_spec((128, 128)),
                  _full_spec((1, 128)), _full_spec((128, 2)), _full_spec((1, 2))],
        out_specs=[_row_spec(2)],
        out_shape=[jax.ShapeDtypeStruct((n_nodes, 2), jnp.float32)],
    )(p, rc, r4, Wl4, Wo1, bo1, Wo2, bo2)[0]


@functools.lru_cache(maxsize=None)
def _make_sc_agg(n_nodes, width, n_edges):
    epc = n_edges // (NC * NS)
    n_ch = epc // CH
    n_rounds = n_ch // NBUF
    npad = -(-n_nodes // (NS * 8)) * (NS * 8)
    rpt = npad // NS
    mesh = plsc.VectorSubcoreMesh(core_axis_name="c", subcore_axis_name="s",
                                  num_cores=NC, num_subcores=NS)

    @functools.partial(
        pl.kernel,
        mesh=mesh,
        out_type=jax.ShapeDtypeStruct((NC, npad, width), jnp.float32),
        scratch_types=(
            [pltpu.VMEM((n_ch, CH), jnp.int32)] * 2
            + [pltpu.VMEM((CH, width), jnp.float32)] * NBUF
            + [pltpu.VMEM_SHARED((npad, width), jnp.float32)]
            + [pltpu.SemaphoreType.DMA] * (NBUF + 1)
        ),
        compiler_params=pltpu.CompilerParams(use_tc_tiling_on_sc=False),
    )
    def agg(x_hbm, edges_hbm, zeros_hbm, out_hbm, *scr):
        sidx, didx = scr[0], scr[1]
        rows = scr[2:2 + NBUF]
        acc = scr[2 + NBUF]
        gsem = scr[3 + NBUF:3 + 2 * NBUF]
        isem = scr[3 + 2 * NBUF]
        c = lax.axis_index("c")
        s = lax.axis_index("s")
        tile = c * NS + s
        chunk0 = tile * n_ch
        row0 = s * rpt

        pltpu.async_copy(edges_hbm.at[0, pl.ds(chunk0, n_ch)], sidx, isem)
        pltpu.async_copy(edges_hbm.at[1, pl.ds(chunk0, n_ch)], didx, isem)
        pltpu.sync_copy(zeros_hbm.at[pl.ds(row0, rpt)], acc.at[pl.ds(row0, rpt)])
        pltpu.make_async_copy(edges_hbm.at[0, pl.ds(chunk0, n_ch)], sidx, isem).wait()
        pltpu.make_async_copy(edges_hbm.at[1, pl.ds(chunk0, n_ch)], didx, isem).wait()
        plsc.subcore_barrier()

        def gather(i, b):
            return pltpu.async_copy(x_hbm.at[sidx.at[i]], rows[b], gsem[b])

        def scatter(i, b):
            pltpu.make_async_copy(x_hbm.at[sidx.at[i]], rows[b], gsem[b]).wait()
            pltpu.sync_copy(rows[b], acc.at[didx.at[i]], add=True)

        for b in range(NBUF):
            gather(b, b)

        def body(r, _):
            i0 = r * NBUF
            for b in range(NBUF):
                scatter(i0 + b, b)
                gather(i0 + NBUF + b, b)
            return 0

        lax.fori_loop(0, n_rounds - 1, body, 0)
        i0 = (n_rounds - 1) * NBUF
        for b in range(NBUF):
            scatter(i0 + b, b)

        plsc.subcore_barrier()
        pltpu.sync_copy(acc.at[pl.ds(row0, rpt)],
                        out_hbm.at[c, pl.ds(row0, rpt)])

    return agg


@functools.lru_cache(maxsize=None)
def _make_sc_agg_split(n_nodes, n_edges):
    half = 32
    eps = n_edges // NS
    n_ch = eps // CH
    n_rounds = n_ch // NBUF
    npad = -(-n_nodes // (NS * 8)) * (NS * 8)
    rpt = npad // NS
    mesh = plsc.VectorSubcoreMesh(core_axis_name="c", subcore_axis_name="s",
                                  num_cores=NC, num_subcores=NS)

    @functools.partial(
        pl.kernel,
        mesh=mesh,
        out_type=jax.ShapeDtypeStruct((npad, 2 * half), jnp.float32),
        scratch_types=(
            [pltpu.VMEM((n_ch, CH), jnp.int32)] * 2
            + [pltpu.VMEM((CH, half), jnp.float32)] * NBUF
            + [pltpu.VMEM_SHARED((npad, half), jnp.float32)]
            + [pltpu.SemaphoreType.DMA] * (NBUF + 1)
        ),
        compiler_params=pltpu.CompilerParams(use_tc_tiling_on_sc=False),
    )
    def agg(x2_hbm, srcab_hbm, edges_hbm, zeros_hbm, out_hbm, *scr):
        sidx, didx = scr[0], scr[1]
        rows = scr[2:2 + NBUF]
        acc = scr[2 + NBUF]
        gsem = scr[3 + NBUF:3 + 2 * NBUF]
        isem = scr[3 + 2 * NBUF]
        c = lax.axis_index("c")
        s = lax.axis_index("s")
        chunk0 = s * n_ch
        row0 = s * rpt

        pltpu.async_copy(srcab_hbm.at[c, pl.ds(chunk0, n_ch)], sidx, isem)
        pltpu.async_copy(edges_hbm.at[1, pl.ds(chunk0, n_ch)], didx, isem)
        pltpu.sync_copy(zeros_hbm.at[pl.ds(row0, rpt)], acc.at[pl.ds(row0, rpt)])
        pltpu.make_async_copy(srcab_hbm.at[c, pl.ds(chunk0, n_ch)], sidx, isem).wait()
        pltpu.make_async_copy(edges_hbm.at[1, pl.ds(chunk0, n_ch)], didx, isem).wait()
        plsc.subcore_barrier()

        def gather(i, b):
            return pltpu.async_copy(x2_hbm.at[sidx.at[i]], rows[b], gsem[b])

        def scatter(i, b):
            pltpu.make_async_copy(x2_hbm.at[sidx.at[i]], rows[b], gsem[b]).wait()
            pltpu.sync_copy(rows[b], acc.at[didx.at[i]], add=True)

        for b in range(NBUF):
            gather(b, b)

        def body(r, _):
            i0 = r * NBUF
            for b in range(NBUF):
                scatter(i0 + b, b)
                gather(i0 + NBUF + b, b)
            return 0

        lax.fori_loop(0, n_rounds - 1, body, 0)
        i0 = (n_rounds - 1) * NBUF
        for b in range(NBUF):
            scatter(i0 + b, b)

        plsc.subcore_barrier()
        pltpu.sync_copy(acc.at[pl.ds(row0, rpt)],
                        out_hbm.at[pl.ds(row0, rpt), pl.ds(c * half, half)])

    return agg


def _sc_agg(x, edges3, srcab=None):
    n_nodes, width = x.shape
    n_edges = edges3.shape[1] * edges3.shape[2]
    npad = -(-n_nodes // (NS * 8)) * (NS * 8)
    if width == 64 and srcab is not None:
        zeros = jnp.zeros((npad, 32), jnp.float32)
        x2 = x.reshape(2 * n_nodes, 32)
        return _make_sc_agg_split(n_nodes, n_edges)(x2, srcab, edges3, zeros)
    zeros = jnp.zeros((npad, width), jnp.float32)
    return _make_sc_agg(n_nodes, width, n_edges)(x, edges3, zeros)


def kernel(des, tweet, num_prop, cat_prop, edge_index, W_des, b_des, W_num,
           b_num, W_cat, b_cat, W_in, b_in, s1a_Wl, s1a_Wr, s1a_b, s1b_Wl,
           s1b_Wr, s1b_b, s2a_Wl, s2a_Wr, s2a_b, s2b_Wl, s2b_Wr, s2b_b,
           W_o1, b_o1, W_o2, b_o2):
    n_nodes = des.shape[0]
    n_edges = edge_index.shape[1]
    edges3 = edge_index.reshape(2, n_edges // CH, CH)
    src2 = edge_index[0] * 2
    srcab = jnp.stack([src2, src2 + 1]).reshape(2, n_edges // CH, CH)

    Wl1p = jnp.concatenate([s1a_Wl, jnp.zeros((128, 16), jnp.float32)], axis=1)
    B1p = jnp.concatenate([jnp.zeros((1, 64), jnp.float32),
                           jnp.ones((1, 16), jnp.float32)], axis=1)

    y1p, r1 = _tc1(
        n_nodes, des, num_prop, cat_prop,
        W_des, b_des.reshape(1, -1), W_num, b_num.reshape(1, -1),
        W_cat, b_cat.reshape(1, -1),
        W_in[:32], W_in[32:74], W_in[74:116], b_in.reshape(1, -1),
        Wl1p, B1p, s1a_Wr, s1a_b.reshape(1, -1))

    p1 = _sc_agg(y1p, edges3)
    h1, r2, rc = _tc2(n_nodes, p1, r1, s1b_Wr, s1b_b.reshape(1, -1))

    p2 = _sc_agg(h1, edges3, srcab)
    y3, r3 = _tc3(n_nodes, p2, rc, r2, s1b_Wl, s2a_Wl, s2a_Wr,
                  s2a_b.reshape(1, -1))

    p3 = _sc_agg(y3, edges3, srcab)
    h3, r4 = _tc4(n_nodes, p3, rc, r3, s2b_Wr, s2b_b.reshape(1, -1))

    p4 = _sc_agg(h3, edges3, srcab)
    out = _tc5(n_nodes, p4, rc, r4, s2b_Wl, W_o1, b_o1.reshape(1, -1),
               W_o2, b_o2.reshape(1, -1))
    return out
